# Initial kernel scaffold; baseline (speedup 1.0000x reference)
import jax
import jax.numpy as jnp
from jax.experimental import pallas as pl


def kernel(x, edge_index, batch, W1l, b1, W1r, W2l, b2, W2r, Wc, bc):
    raise NotImplementedError("write your pallas kernel here")


# speedup vs baseline: 20.1313x; 20.1313x over previous
import functools

import jax
import jax.numpy as jnp
from jax import lax
from jax.experimental import pallas as pl
from jax.experimental.pallas import tpu as pltpu
from jax.experimental.pallas import tpu_sc as plsc

N = 100000
E = 3200000
H = 64
G = 256
NC, NS = 2, 16
NW = NC * NS
EPW = E // NW
RW = 100
K = 8
CH = K * RW
NCHUNK = EPW // CH
NP = 100352
RPT = NP // NS
BLK = 2000
NBLK = N // BLK

_mesh = plsc.VectorSubcoreMesh(core_axis_name="c", subcore_axis_name="s")


@functools.partial(
    pl.kernel,
    out_type=jax.ShapeDtypeStruct((NC, NP, 16), jnp.float32),
    mesh=_mesh,
    scratch_types=[
        pltpu.VMEM((K, RW), jnp.int32),
        pltpu.VMEM((K, RW), jnp.int32),
        pltpu.VMEM((CH, 16), jnp.float32),
        pltpu.VMEM_SHARED((NP, 16), jnp.float32),
        pltpu.SemaphoreType.DMA,
        pltpu.SemaphoreType.DMA,
    ],
    compiler_params=pltpu.CompilerParams(use_tc_tiling_on_sc=False),
)
def _edge_pass(table_h, src_h, dst_h, zeros_h, out_h,
               src_v, dst_v, rows_v, acc, gsem, ssem):
    cid = lax.axis_index("c")
    sid = lax.axis_index("s")
    w = sid * NC + cid
    pltpu.sync_copy(zeros_h.at[pl.ds(sid * RPT, RPT)],
                    acc.at[pl.ds(sid * RPT, RPT)])
    plsc.subcore_barrier()

    def chunk(c, carry):
        row0 = w * (EPW // RW) + c * K
        cp_s = pltpu.async_copy(src_h.at[pl.ds(row0, K)], src_v, gsem)
        cp_d = pltpu.async_copy(dst_h.at[pl.ds(row0, K)], dst_v, ssem)
        cp_s.wait()
        cp_d.wait()
        gathers = [
            pltpu.async_copy(table_h.at[src_v.at[j]],
                             rows_v.at[pl.ds(j * RW, RW)], gsem)
            for j in range(K)
        ]
        for g in gathers:
            g.wait()
        scatters = [
            pltpu.async_copy(rows_v.at[pl.ds(j * RW, RW)],
                             acc.at[dst_v.at[j]], ssem, add=True)
            for j in range(K)
        ]
        for s in scatters:
            s.wait()
        return carry

    lax.fori_loop(0, NCHUNK, chunk, 0)
    plsc.subcore_barrier()
    pltpu.sync_copy(acc.at[pl.ds(sid * RPT, RPT)],
                    out_h.at[cid, pl.ds(sid * RPT, RPT)])


def _mid_body(pa_ref, x_ref, w1l_ref, w1r_ref, b1_ref, h1g_ref, cnt_ref):
    s = pa_ref[0] + pa_ref[1]
    sum1 = s[:, 0:1]
    cntb = s[:, 1:2]
    m = sum1 / jnp.maximum(cntb, 1.0)
    h1 = jnp.maximum(m * w1l_ref[...] + x_ref[...] * w1r_ref[...]
                     + b1_ref[...], 0.0)
    for f in range(4):
        h1g_ref[f] = h1[:, f * 16:(f + 1) * 16]
    cnt_ref[...] = cntb


def _tc_mid(pa, x, w1l, w1r, b1):
    return pl.pallas_call(
        _mid_body,
        grid=(NBLK,),
        in_specs=[
            pl.BlockSpec((2, BLK, 16), lambda i: (0, i, 0)),
            pl.BlockSpec((BLK, 1), lambda i: (i, 0)),
            pl.BlockSpec((1, H), lambda i: (0, 0)),
            pl.BlockSpec((1, H), lambda i: (0, 0)),
            pl.BlockSpec((1, H), lambda i: (0, 0)),
        ],
        out_specs=[
            pl.BlockSpec((4, BLK, 16), lambda i: (0, i, 0)),
            pl.BlockSpec((BLK, 1), lambda i: (i, 0)),
        ],
        out_shape=[
            jax.ShapeDtypeStruct((4, N, 16), jnp.float32),
            jax.ShapeDtypeStruct((N, 1), jnp.float32),
        ],
        compiler_params=pltpu.CompilerParams(
            dimension_semantics=("arbitrary",)),
    )(pa, x, w1l, w1r, b1)


def _final_body(pb0_ref, pb1_ref, pb2_ref, pb3_ref, h1g_ref, cnt_ref,
                batch_ref, w2l_ref, w2r_ref, b2_ref, wc_ref, bc_ref,
                out_ref, accs):
    i = pl.program_id(0)

    @pl.when(i == 0)
    def _():
        accs[...] = jnp.zeros((G, 128), jnp.float32)

    sum2 = jnp.concatenate(
        [p[0] + p[1] for p in (pb0_ref, pb1_ref, pb2_ref, pb3_ref)], axis=1)
    h1 = jnp.concatenate([h1g_ref[f] for f in range(4)], axis=1)
    cntc = jnp.maximum(cnt_ref[...], 1.0)
    mean2 = sum2 / cntc
    h2 = jnp.maximum(
        jax.lax.dot(mean2, w2l_ref[...], precision=lax.Precision.HIGHEST)
        + jax.lax.dot(h1, w2r_ref[...], precision=lax.Precision.HIGHEST)
        + b2_ref[...], 0.0)
    h2e = jnp.concatenate(
        [h2, jnp.ones((BLK, 1), jnp.float32),
         jnp.zeros((BLK, 63), jnp.float32)], axis=1)
    oh = (batch_ref[...] ==
          lax.broadcasted_iota(jnp.int32, (1, G), 1)).astype(jnp.float32)
    accs[...] += lax.dot_general(oh, h2e, (((0,), (0,)), ((), ())),
                                 precision=lax.Precision.HIGHEST)

    @pl.when(i == NBLK - 1)
    def _():
        pooled = accs[:, 0:H] / jnp.maximum(accs[:, H:H + 1], 1.0)
        out_ref[...] = (
            jax.lax.dot(pooled, wc_ref[...], precision=lax.Precision.HIGHEST)
            + bc_ref[...])


def _tc_final(pbs, h1g, cnt, batch2, w2l, w2r, b2, wc, bc):
    c = wc.shape[1]
    return pl.pallas_call(
        _final_body,
        grid=(NBLK,),
        in_specs=[
            pl.BlockSpec((2, BLK, 16), lambda i: (0, i, 0)),
            pl.BlockSpec((2, BLK, 16), lambda i: (0, i, 0)),
            pl.BlockSpec((2, BLK, 16), lambda i: (0, i, 0)),
            pl.BlockSpec((2, BLK, 16), lambda i: (0, i, 0)),
            pl.BlockSpec((4, BLK, 16), lambda i: (0, i, 0)),
            pl.BlockSpec((BLK, 1), lambda i: (i, 0)),
            pl.BlockSpec((BLK, 1), lambda i: (i, 0)),
            pl.BlockSpec((H, H), lambda i: (0, 0)),
            pl.BlockSpec((H, H), lambda i: (0, 0)),
            pl.BlockSpec((1, H), lambda i: (0, 0)),
            pl.BlockSpec((H, c), lambda i: (0, 0)),
            pl.BlockSpec((1, c), lambda i: (0, 0)),
        ],
        out_specs=pl.BlockSpec((G, c), lambda i: (0, 0)),
        out_shape=jax.ShapeDtypeStruct((G, c), jnp.float32),
        scratch_shapes=[pltpu.VMEM((G, 128), jnp.float32)],
        compiler_params=pltpu.CompilerParams(
            dimension_semantics=("arbitrary",)),
    )(*pbs, h1g, cnt, batch2, w2l, w2r, b2, wc, bc)


def kernel(x, edge_index, batch, W1l, b1, W1r, W2l, b2, W2r, Wc, bc):
    src2 = edge_index[0].reshape(E // RW, RW)
    dst2 = edge_index[1].reshape(E // RW, RW)
    zeros_pad = jnp.zeros((NP, 16), jnp.float32)
    xp = jnp.concatenate(
        [x, jnp.ones_like(x), jnp.zeros((N, 14), jnp.float32)], axis=1)
    pa = _edge_pass(xp, src2, dst2, zeros_pad)

    h1g, cnt = _tc_mid(pa, x, W1l.reshape(1, H), W1r.reshape(1, H),
                       b1.reshape(1, H))

    pbs = [_edge_pass(h1g[f], src2, dst2, zeros_pad) for f in range(4)]

    return _tc_final(pbs, h1g, cnt, batch.reshape(N, 1),
                     W2l, W2r, b2.reshape(1, H), Wc, bc.reshape(1, Wc.shape[1]))
